# baseline (speedup 1.0000x reference)
import functools

import jax
import jax.numpy as jnp
from jax import lax
from jax.experimental import pallas as pl
from jax.experimental.pallas import tpu as pltpu
from jax.experimental.pallas import tpu_sc as plsc

D = 64
SCALE = float(D) ** 0.5

NC = 2
NS = 16
NW = NC * NS
L = 16
BW = 128


def _body(n_t, table_hbm, idx_hbm, out_hbm,
          idx_v, pair_v, half_v, rows_v, trans_v, sem_rows, sem_out):
    wid = lax.axis_index("s") * NC + lax.axis_index("c")
    b0 = wid * BW

    pltpu.sync_copy(idx_hbm.at[:, pl.ds(b0, BW)], idx_v)

    def gather_copy(buf):
        return pltpu.make_async_copy(table_hbm.at[pair_v.at[buf]],
                                     rows_v.at[buf], sem_rows)

    def out_copy(t, buf):
        return pltpu.make_async_copy(
            trans_v.at[buf], out_hbm.at[t, :, pl.ds(b0, BW)], sem_out)

    def prep_indices(t, buf):
        for j in range(BW // L):
            v = idx_v[t, pl.ds(j * L, L)]
            pair_v.at[buf][pl.ds(j * L, L)] = lax.shift_right_logical(v, 1)
            half_v.at[buf][pl.ds(j * L, L)] = (v & 1) * D

    def transpose_scale(buf):
        lanes = lax.iota(jnp.int32, L)

        def d_fn(d, _):
            for j in range(BW // L):
                row = lanes + (j * L)
                col = half_v.at[buf][pl.ds(j * L, L)] + d
                vals = plsc.load_gather(rows_v.at[buf], [row, col])
                trans_v.at[buf][d, pl.ds(j * L, L)] = vals * SCALE
            return 0
        lax.fori_loop(0, D, d_fn, 0, unroll=4)

    prep_indices(0, 0)
    gather_copy(0).start()

    def step(t, _):
        buf = lax.rem(t, 2)
        nxt = 1 - buf

        @pl.when(t + 1 < n_t)
        def _():
            prep_indices(t + 1, nxt)
            gather_copy(nxt).start()

        gather_copy(buf).wait()

        @pl.when(t >= 2)
        def _():
            out_copy(t - 2, buf).wait()

        transpose_scale(buf)
        out_copy(t, buf).start()
        return 0

    lax.fori_loop(0, n_t, step, 0)
    out_copy(n_t - 2, lax.rem(n_t, 2)).wait()
    out_copy(n_t - 1, lax.rem(n_t - 1, 2)).wait()


@functools.partial(jax.jit, static_argnames=("n_t",))
def _embed_sc(idx_t, table2, n_t):
    mesh = plsc.VectorSubcoreMesh(core_axis_name="c", subcore_axis_name="s")
    run = pl.kernel(
        functools.partial(_body, n_t),
        out_type=jax.ShapeDtypeStruct((n_t, D, NW * BW), jnp.float32),
        mesh=mesh,
        scratch_types=[
            pltpu.VMEM((n_t, BW), jnp.int32),
            pltpu.VMEM((2, BW), jnp.int32),
            pltpu.VMEM((2, BW), jnp.int32),
            pltpu.VMEM((2, BW, 2 * D), jnp.float32),
            pltpu.VMEM((2, D, BW), jnp.float32),
            pltpu.SemaphoreType.DMA,
            pltpu.SemaphoreType.DMA,
        ],
        compiler_params=pltpu.CompilerParams(use_tc_tiling_on_sc=True,
                                             needs_layout_passes=False),
    )
    return run(table2, idx_t)


def kernel(inputs, lookup_table):
    B, T = inputs.shape
    idx_t = inputs.T.astype(jnp.int32)
    table2 = lookup_table.reshape(-1, 2 * D)
    out_t = _embed_sc(idx_t, table2, T)
    return jnp.transpose(out_t, (2, 0, 1))


# speedup vs baseline: 2.6048x; 2.6048x over previous
import functools

import jax
import jax.numpy as jnp
from jax import lax
from jax.experimental import pallas as pl
from jax.experimental.pallas import tpu as pltpu
from jax.experimental.pallas import tpu_sc as plsc

D = 64
W = 2 * D
SCALE = float(D) ** 0.5

NC = 2
NS = 16
NW = NC * NS
L = 16
G = 128
CHUNK = 256


def _body(n_chunks, table_hbm, idx_hbm, out_hbm,
          idx_v, rows_v, sem_rows, sem_out):
    wid = lax.axis_index("s") * NC + lax.axis_index("c")
    b_per_w = CHUNK * n_chunks
    base = wid * b_per_w
    K = CHUNK // G

    row0 = pl.multiple_of(base // G, 8)
    pltpu.sync_copy(idx_hbm.at[pl.ds(row0, b_per_w // G), :], idx_v)

    def gather(buf, g, do_start):
        for k in range(K):
            cp = pltpu.make_async_copy(
                table_hbm.at[idx_v.at[g * K + k]],
                rows_v.at[buf, pl.ds(k * G, G), :], sem_rows)
            if do_start:
                cp.start()
            else:
                cp.wait()

    def out_copy(g, buf):
        return pltpu.make_async_copy(
            rows_v.at[buf],
            out_hbm.at[pl.ds(pl.multiple_of(base + g * CHUNK, CHUNK), CHUNK)],
            sem_out)

    def scale_rows(buf):
        rb = rows_v.at[buf]

        def row_fn(r, _):
            for j in range(D // L):
                sl = (r, pl.ds(j * L, L))
                rb[sl] = rb[sl] * SCALE
            return 0
        lax.fori_loop(0, CHUNK, row_fn, 0, unroll=4)

    gather(0, 0, True)

    def step(g, _):
        buf = lax.rem(g, 2)
        nxt = 1 - buf

        @pl.when(g >= 1)
        def _():
            out_copy(g - 1, nxt).wait()

        gather(buf, g, False)

        @pl.when(g + 1 < n_chunks)
        def _():
            gather(nxt, g + 1, True)

        scale_rows(buf)
        out_copy(g, buf).start()
        return 0

    lax.fori_loop(0, n_chunks, step, 0)
    out_copy(n_chunks - 1, lax.rem(n_chunks - 1, 2)).wait()


@functools.partial(jax.jit, static_argnames=("n_chunks",))
def _embed_sc(idx2, table_wide, n_chunks):
    b_total = CHUNK * n_chunks * NW
    mesh = plsc.VectorSubcoreMesh(core_axis_name="c", subcore_axis_name="s")
    run = pl.kernel(
        functools.partial(_body, n_chunks),
        out_type=jax.ShapeDtypeStruct((b_total, W), jnp.float32),
        mesh=mesh,
        scratch_types=[
            pltpu.VMEM((CHUNK * n_chunks // G, G), jnp.int32),
            pltpu.VMEM((2, CHUNK, W), jnp.float32),
            pltpu.SemaphoreType.DMA,
            pltpu.SemaphoreType.DMA,
        ],
        compiler_params=pltpu.CompilerParams(use_tc_tiling_on_sc=True,
                                             needs_layout_passes=False),
    )
    return run(table_wide, idx2)


def kernel(inputs, lookup_table):
    B, T = inputs.shape
    idx2 = inputs.reshape(B * T // G, G).astype(jnp.int32)
    table_wide = jnp.pad(lookup_table, ((0, 0), (0, W - D)))
    b_per_w = (B * T) // NW
    out = _embed_sc(idx2, table_wide, b_per_w // CHUNK)
    return out[:, :D].reshape(B, T, D)
